# baseline (speedup 1.0000x reference)
import jax
import jax.numpy as jnp
from jax.experimental import pallas as pl
from jax.experimental.pallas import tpu as pltpu

_W = 224
_ROWS = 16 * 96 * 224
_B = 14336
_GRID = _ROWS // _B
_CH = 64


def _loss_kernel(out_ref, tgt_ref, res_ref, acc_ref):
    i = pl.program_id(0)
    ones = jnp.ones((8, _CH), jnp.float32)
    acc = None
    for r in range(0, _B, _CH):
        o = out_ref[pl.ds(r, _CH), :]
        t = tgt_ref[pl.ds(r, _CH), :]
        d = o - t
        a = jnp.abs(d)
        m = jnp.minimum(a, 1.0)
        f = m * (a - 0.5 * m)
        f = jnp.where(t != 0.0, f, 0.0)
        p = jax.lax.dot_general(
            ones, f, (((1,), (0,)), ((), ())),
            preferred_element_type=jnp.float32)
        acc = p if acc is None else acc + p

    @pl.when(i == 0)
    def _init():
        acc_ref[...] = acc

    @pl.when(i > 0)
    def _acc():
        acc_ref[...] = acc_ref[...] + acc

    @pl.when(i == _GRID - 1)
    def _fin():
        res_ref[...] = (jnp.sum(acc_ref[...]) * 0.125)[None, None]


def kernel(out, target):
    o2 = out.reshape(_ROWS, _W)
    t2 = target.reshape(_ROWS, _W)
    res = pl.pallas_call(
        _loss_kernel,
        grid=(_GRID,),
        in_specs=[
            pl.BlockSpec((_B, _W), lambda i: (i, 0)),
            pl.BlockSpec((_B, _W), lambda i: (i, 0)),
        ],
        out_specs=pl.BlockSpec((1, 1), lambda i: (0, 0)),
        out_shape=jax.ShapeDtypeStruct((1, 1), jnp.float32),
        scratch_shapes=[pltpu.VMEM((8, _W), jnp.float32)],
        compiler_params=pltpu.CompilerParams(
            dimension_semantics=("arbitrary",),
            vmem_limit_bytes=63 * 1024 * 1024,
        ),
    )(o2, t2)
    return res[0, 0]


# speedup vs baseline: 1.0113x; 1.0113x over previous
import jax
import jax.numpy as jnp
from jax.experimental import pallas as pl
from jax.experimental.pallas import tpu as pltpu

_W = 224
_ROWS = 16 * 96 * 224
_B = 12288
_GRID = _ROWS // _B
_CH = 64


def _loss_kernel(out_ref, tgt_ref, res_ref, acc_ref):
    i = pl.program_id(0)
    acc = None
    for r in range(0, _B, _CH):
        o = out_ref[pl.ds(r, _CH), :]
        t = tgt_ref[pl.ds(r, _CH), :]
        d = o - t
        a = jnp.abs(d)
        m = jnp.minimum(a, 1.0)
        f = m * (a - 0.5 * m)
        f = jnp.where(t != 0.0, f, 0.0)
        p = jnp.sum(f.reshape(-1, 8, _W), axis=0)
        acc = p if acc is None else acc + p

    @pl.when(i == 0)
    def _init():
        acc_ref[...] = acc

    @pl.when(i > 0)
    def _acc():
        acc_ref[...] = acc_ref[...] + acc

    @pl.when(i == _GRID - 1)
    def _fin():
        res_ref[...] = jnp.sum(acc_ref[...])[None, None]


def kernel(out, target):
    o2 = out.reshape(_ROWS, _W)
    t2 = target.reshape(_ROWS, _W)
    res = pl.pallas_call(
        _loss_kernel,
        grid=(_GRID,),
        in_specs=[
            pl.BlockSpec((_B, _W), lambda i: (i, 0)),
            pl.BlockSpec((_B, _W), lambda i: (i, 0)),
        ],
        out_specs=pl.BlockSpec((1, 1), lambda i: (0, 0)),
        out_shape=jax.ShapeDtypeStruct((1, 1), jnp.float32),
        scratch_shapes=[pltpu.VMEM((8, _W), jnp.float32)],
        compiler_params=pltpu.CompilerParams(
            dimension_semantics=("arbitrary",),
            vmem_limit_bytes=63 * 1024 * 1024,
        ),
    )(o2, t2)
    return res[0, 0]
